# baseline (speedup 1.0000x reference)
import functools

import jax
import jax.numpy as jnp
from jax import lax
from jax.experimental import pallas as pl
from jax.experimental.pallas import tpu as pltpu
from jax.experimental.pallas import tpu_sc as plsc

N_FEATURES = 26
FEATURE_SIZE = 100000
EMB_DIM = 32
BATCH = 16384
TOTAL = BATCH * N_FEATURES
SUPER = 128 // EMB_DIM

_INFO = plsc.get_sparse_core_info()
NC = _INFO.num_cores
NS = _INFO.num_subcores
NW = NC * NS
PER_W = TOTAL // NW
ROW = 128
G = PER_W // ROW


@functools.partial(
    pl.kernel,
    mesh=plsc.VectorSubcoreMesh(core_axis_name="c", subcore_axis_name="s"),
    out_type=jax.ShapeDtypeStruct((TOTAL, EMB_DIM), jnp.float32),
    scratch_types=[
        pltpu.VMEM((G, ROW), jnp.int32),
        pltpu.VMEM((G, ROW), jnp.int32),
        pltpu.VMEM((ROW, 128), jnp.float32),
        pltpu.VMEM((ROW, EMB_DIM), jnp.float32),
        pltpu.SemaphoreType.DMA,
    ],
    compiler_params=pltpu.CompilerParams(
        use_tc_tiling_on_sc=True, needs_layout_passes=False
    ),
)
def _gather_kernel(ids_hbm, table_hbm, out_hbm, sup_v, col_v, rows_v, o_v, sem):
    wid = lax.axis_index("s") * NC + lax.axis_index("c")
    row0 = wid * G
    pltpu.sync_copy(ids_hbm.at[pl.ds(row0, G)], sup_v)

    base = row0 * ROW
    iota = lax.iota(jnp.int32, 16)

    def fix_row(g, carry):
        for s in range(ROW // 16):
            sl = pl.ds(s * 16, 16)
            jv = (base + g * ROW + s * 16) + iota
            full = sup_v[g, sl] + (jv % N_FEATURES) * FEATURE_SIZE
            sup_v[g, sl] = lax.shift_right_logical(full, 2)
            col_v[g, sl] = (full & 3) * EMB_DIM
        return carry

    lax.fori_loop(0, G, fix_row, 0)

    def gather_row(g, carry):
        pltpu.async_copy(table_hbm.at[sup_v.at[g]], rows_v, sem).wait()
        for j in range(ROW // 16):
            rows = j * 16 + iota
            cb = col_v[g, pl.ds(j * 16, 16)]
            for c in range(EMB_DIM):
                v = plsc.load_gather(rows_v, [rows, cb + c])
                plsc.store_scatter(o_v, [rows, jnp.full((16,), c, jnp.int32)], v)
        pltpu.sync_copy(o_v, out_hbm.at[pl.ds((row0 + g) * ROW, ROW)])
        return carry

    lax.fori_loop(0, G, gather_row, 0)


def kernel(feature_ids, embed_weight):
    ids2 = feature_ids.reshape(TOTAL // ROW, ROW)
    table128 = embed_weight.reshape(N_FEATURES * FEATURE_SIZE // SUPER, 128)
    out = _gather_kernel(ids2, table128)
    return out.reshape(BATCH, N_FEATURES, EMB_DIM)
